# baseline (speedup 1.0000x reference)
import functools

import jax
import jax.numpy as jnp
from jax import lax
from jax.experimental import pallas as pl
from jax.experimental.pallas import tpu as pltpu
from jax.experimental.pallas import tpu_sc as plsc

LANES = 16


@functools.lru_cache(maxsize=None)
def _build_sc_call(n_rows: int, emb_dim: int):
    NW = 32
    per_w = n_rows // NW
    chunk = 128
    blk = 512
    gpb = blk // chunk
    n_blk = per_w // blk
    assert per_w * NW == n_rows and n_blk * blk == per_w and n_blk % 2 == 0

    mesh = plsc.VectorSubcoreMesh(core_axis_name="c", subcore_axis_name="s")

    @functools.partial(
        pl.kernel,
        out_type=jax.ShapeDtypeStruct((n_rows, emb_dim), jnp.float32),
        mesh=mesh,
        compiler_params=pltpu.CompilerParams(use_tc_tiling_on_sc=False),
        scratch_types=[
            pltpu.VMEM((per_w,), jnp.int32),
            pltpu.VMEM((per_w,), jnp.int32),
            pltpu.VMEM((blk, emb_dim), jnp.float32),
            pltpu.VMEM((blk, emb_dim), jnp.float32),
            pltpu.VMEM((blk, emb_dim), jnp.float32),
            pltpu.VMEM((blk, emb_dim), jnp.float32),
            pltpu.VMEM((blk, emb_dim), jnp.float32),
            pltpu.VMEM((blk, emb_dim), jnp.float32),
            pltpu.SemaphoreType.DMA,
            pltpu.SemaphoreType.DMA,
            pltpu.SemaphoreType.DMA,
            pltpu.SemaphoreType.DMA,
        ],
    )
    def sc_call(idx1_hbm, idx2_hbm, emb1_hbm, emb2_hbm, out_hbm,
                idx1_v, idx2_v, r1a, r1b, r2a, r2b, pa, pb,
                sg_a, sg_b, so_a, so_b):
        r1 = (r1a, r1b)
        r2 = (r2a, r2b)
        prod = (pa, pb)
        sg = (sg_a, sg_b)
        so = (so_a, so_b)

        wid = lax.axis_index("s") * 2 + lax.axis_index("c")
        base = wid * per_w
        pltpu.sync_copy(idx1_hbm.at[pl.ds(base, per_w)], idx1_v)
        pltpu.sync_copy(idx2_hbm.at[pl.ds(base, per_w)], idx2_v)

        def fire_gathers(j, slot):
            off = j * blk
            for g in range(gpb):
                o = off + g * chunk
                dst = pl.ds(g * chunk, chunk)
                pltpu.async_copy(
                    emb1_hbm.at[idx1_v.at[pl.ds(o, chunk)]],
                    r1[slot].at[dst], sg[slot])
                pltpu.async_copy(
                    emb2_hbm.at[idx2_v.at[pl.ds(o, chunk)]],
                    r2[slot].at[dst], sg[slot])

        def drain_gathers(slot):
            pltpu.make_async_copy(
                emb1_hbm.at[pl.ds(0, blk)], r1[slot], sg[slot]).wait()
            pltpu.make_async_copy(
                emb1_hbm.at[pl.ds(0, blk)], r2[slot], sg[slot]).wait()

        def drain_out(slot):
            pltpu.make_async_copy(
                prod[slot], out_hbm.at[pl.ds(0, blk)], so[slot]).wait()

        def multiply(slot):
            a, b, p = r1[slot], r2[slot], prod[slot]

            def mul_body(r, c):
                for h in range(emb_dim // LANES):
                    sl = pl.ds(h * LANES, LANES)
                    p[r, sl] = a[r, sl] * b[r, sl]
                return c

            lax.fori_loop(0, blk, mul_body, 0, unroll=8)

        fire_gathers(0, 0)
        fire_gathers(1, 1)

        def step(i, carry):
            for slot in range(2):
                j = 2 * i + slot
                drain_gathers(slot)

                @pl.when(i > 0)
                def _():
                    drain_out(slot)

                multiply(slot)

                @pl.when(j + 2 < n_blk)
                def _():
                    fire_gathers(j + 2, slot)

                pltpu.async_copy(
                    prod[slot], out_hbm.at[pl.ds(base + j * blk, blk)],
                    so[slot])
            return carry

        lax.fori_loop(0, n_blk // 2, step, 0)
        drain_out(0)
        drain_out(1)

    return sc_call


@functools.lru_cache(maxsize=None)
def _build_transpose_call(n_tab: int, emb_dim: int):
    NW = 32
    mesh = plsc.VectorSubcoreMesh(core_axis_name="c", subcore_axis_name="s")
    rows_per_srow = 128 // emb_dim
    s_rows = n_tab // rows_per_srow
    n_full = n_tab // 128
    rem = n_tab - n_full * 128
    tail_srows = rem // rows_per_srow

    @functools.partial(
        pl.kernel,
        out_type=(jax.ShapeDtypeStruct((s_rows, 128), jnp.float32),
                  jax.ShapeDtypeStruct((s_rows, 128), jnp.float32)),
        mesh=mesh,
        compiler_params=pltpu.CompilerParams(use_tc_tiling_on_sc=True,
                                             needs_layout_passes=False),
        scratch_types=[
            pltpu.VMEM((emb_dim, 128), jnp.float32),
            pltpu.VMEM((emb_dim, 128), jnp.float32),
            pltpu.VMEM((32, 128), jnp.float32),
            pltpu.VMEM((32, 128), jnp.float32),
            pltpu.SemaphoreType.DMA,
            pltpu.SemaphoreType.DMA,
        ],
    )
    def tr_call(e1t_hbm, e2t_hbm, tail1_hbm, tail2_hbm, s1_hbm, s2_hbm,
                src1_v, src2_v, dst1_v, dst2_v, sin, sout):
        wid = lax.axis_index("s") * 2 + lax.axis_index("c")
        e_lo = jax.lax.iota(jnp.int32, 16)

        def transpose_block(src, dst, width):
            def body(rl, c):
                x = rl // 4
                co = (rl % 4) * emb_dim
                rv = jnp.full((16,), rl, jnp.int32)
                g0 = plsc.load_gather(src, [e_lo, rv])
                g1 = plsc.load_gather(src, [e_lo + 16, rv])
                dst[x, pl.ds(co, 16)] = g0
                dst[x, pl.ds(co + 16, 16)] = g1
                return c

            lax.fori_loop(0, width, body, 0, unroll=4)

        def do_block(tr, width, s_cnt):
            cp1 = pltpu.async_copy(
                e1t_hbm.at[pl.ds(0, emb_dim), pl.ds(tr * 128, width)],
                src1_v.at[pl.ds(0, emb_dim), pl.ds(0, width)], sin)
            cp2 = pltpu.async_copy(
                e2t_hbm.at[pl.ds(0, emb_dim), pl.ds(tr * 128, width)],
                src2_v.at[pl.ds(0, emb_dim), pl.ds(0, width)], sin)
            cp1.wait()
            transpose_block(src1_v, dst1_v, width)
            o1 = pltpu.async_copy(
                dst1_v.at[pl.ds(0, s_cnt)], s1_hbm.at[pl.ds(tr * 32, s_cnt)],
                sout)
            cp2.wait()
            transpose_block(src2_v, dst2_v, width)
            o2 = pltpu.async_copy(
                dst2_v.at[pl.ds(0, s_cnt)], s2_hbm.at[pl.ds(tr * 32, s_cnt)],
                sout)
            o1.wait()
            o2.wait()

        n_k = (n_full - wid + NW - 1) // NW

        def body_k(k, c):
            do_block(wid + k * NW, 128, 32)
            return c

        lax.fori_loop(0, n_k, body_k, 0)

        if rem:
            @pl.when(wid == 0)
            def _():
                pltpu.sync_copy(tail1_hbm,
                                src1_v.at[pl.ds(0, tail_srows)])
                pltpu.sync_copy(src1_v.at[pl.ds(0, tail_srows)],
                                s1_hbm.at[pl.ds(n_full * 32, tail_srows)])

            @pl.when(wid == 1)
            def _():
                pltpu.sync_copy(tail2_hbm,
                                src2_v.at[pl.ds(0, tail_srows)])
                pltpu.sync_copy(src2_v.at[pl.ds(0, tail_srows)],
                                s2_hbm.at[pl.ds(n_full * 32, tail_srows)])

    return tr_call


def kernel(first, second, emb1, emb2):
    b, f = first.shape
    emb_dim = emb1.shape[1]
    n_tab = emb1.shape[0]
    n_rows = b * f
    idx1 = first.reshape(n_rows).astype(jnp.int32)
    idx2 = second.reshape(n_rows).astype(jnp.int32)
    tr_call = _build_transpose_call(n_tab, emb_dim)
    n_main = (n_tab // 128) * 128
    tail1 = emb1[n_main:].reshape(-1, 128)
    tail2 = emb2[n_main:].reshape(-1, 128)
    s1, s2 = tr_call(emb1.T, emb2.T, tail1, tail2)
    e1 = s1.reshape(n_tab, emb_dim)
    e2 = s2.reshape(n_tab, emb_dim)
    sc_call = _build_sc_call(n_rows, emb_dim)
    out = sc_call(idx1, idx2, e1, e2)
    return out.reshape(b, f, emb_dim)


# speedup vs baseline: 2.1136x; 2.1136x over previous
import functools

import jax
import jax.numpy as jnp
from jax import lax
from jax.experimental import pallas as pl
from jax.experimental.pallas import tpu as pltpu
from jax.experimental.pallas import tpu_sc as plsc

LANES = 16


@functools.lru_cache(maxsize=None)
def _build_sc_call(n_rows: int, emb_dim: int):
    NW = 32
    per_w = n_rows // NW
    chunk = 128
    blk = 512
    gpb = blk // chunk
    n_blk = per_w // blk
    assert per_w * NW == n_rows and n_blk * blk == per_w and n_blk % 2 == 0

    mesh = plsc.VectorSubcoreMesh(core_axis_name="c", subcore_axis_name="s")

    @functools.partial(
        pl.kernel,
        out_type=jax.ShapeDtypeStruct((n_rows, emb_dim), jnp.float32),
        mesh=mesh,
        compiler_params=pltpu.CompilerParams(use_tc_tiling_on_sc=False),
        scratch_types=[
            pltpu.VMEM((per_w,), jnp.int32),
            pltpu.VMEM((per_w,), jnp.int32),
            pltpu.VMEM((blk, emb_dim), jnp.float32),
            pltpu.VMEM((blk, emb_dim), jnp.float32),
            pltpu.VMEM((blk, emb_dim), jnp.float32),
            pltpu.VMEM((blk, emb_dim), jnp.float32),
            pltpu.VMEM((blk, emb_dim), jnp.float32),
            pltpu.VMEM((blk, emb_dim), jnp.float32),
            pltpu.SemaphoreType.DMA,
            pltpu.SemaphoreType.DMA,
            pltpu.SemaphoreType.DMA,
            pltpu.SemaphoreType.DMA,
        ],
    )
    def sc_call(idx1_hbm, idx2_hbm, emb1_hbm, emb2_hbm, out_hbm,
                idx1_v, idx2_v, r1a, r1b, r2a, r2b, pa, pb,
                sg_a, sg_b, so_a, so_b):
        r1 = (r1a, r1b)
        r2 = (r2a, r2b)
        prod = (pa, pb)
        sg = (sg_a, sg_b)
        so = (so_a, so_b)

        wid = lax.axis_index("s") * 2 + lax.axis_index("c")
        base = wid * per_w
        pltpu.sync_copy(idx1_hbm.at[pl.ds(base, per_w)], idx1_v)
        pltpu.sync_copy(idx2_hbm.at[pl.ds(base, per_w)], idx2_v)

        def fire_gathers(j, slot):
            off = j * blk
            for g in range(gpb):
                o = off + g * chunk
                dst = pl.ds(g * chunk, chunk)
                pltpu.async_copy(
                    emb1_hbm.at[idx1_v.at[pl.ds(o, chunk)]],
                    r1[slot].at[dst], sg[slot])
                pltpu.async_copy(
                    emb2_hbm.at[idx2_v.at[pl.ds(o, chunk)]],
                    r2[slot].at[dst], sg[slot])

        def drain_gathers(slot):
            pltpu.make_async_copy(
                emb1_hbm.at[pl.ds(0, blk)], r1[slot], sg[slot]).wait()
            pltpu.make_async_copy(
                emb1_hbm.at[pl.ds(0, blk)], r2[slot], sg[slot]).wait()

        def drain_out(slot):
            pltpu.make_async_copy(
                prod[slot], out_hbm.at[pl.ds(0, blk)], so[slot]).wait()

        def multiply(slot):
            a, b, p = r1[slot], r2[slot], prod[slot]

            def mul_body(r, c):
                for h in range(emb_dim // LANES):
                    sl = pl.ds(h * LANES, LANES)
                    p[r, sl] = a[r, sl] * b[r, sl]
                return c

            lax.fori_loop(0, blk, mul_body, 0, unroll=8)

        fire_gathers(0, 0)
        fire_gathers(1, 1)

        def step(i, carry):
            for slot in range(2):
                j = 2 * i + slot
                drain_gathers(slot)

                @pl.when(i > 0)
                def _():
                    drain_out(slot)

                multiply(slot)

                @pl.when(j + 2 < n_blk)
                def _():
                    fire_gathers(j + 2, slot)

                pltpu.async_copy(
                    prod[slot], out_hbm.at[pl.ds(base + j * blk, blk)],
                    so[slot])
            return carry

        lax.fori_loop(0, n_blk // 2, step, 0)
        drain_out(0)
        drain_out(1)

    return sc_call


TR_C = 2048


@functools.lru_cache(maxsize=None)
def _build_transpose_call(n_tab: int, emb_dim: int):
    rows_per_srow = 128 // emb_dim
    C = TR_C
    grid = (n_tab + 4 * C - 1) // (4 * C)
    s_rows = grid * C
    max_blk = (n_tab + C - 1) // C - 1

    def body(*refs):
        in_refs, out_ref = refs[:rows_per_srow], refs[rows_per_srow]
        for j in range(rows_per_srow):
            out_ref[:, j * emb_dim:(j + 1) * emb_dim] = jnp.transpose(
                in_refs[j][...])

    return pl.pallas_call(
        body,
        grid=(grid,),
        in_specs=[
            pl.BlockSpec((emb_dim, C),
                         lambda i, j=j: (0, jnp.minimum(4 * i + j, max_blk)))
            for j in range(rows_per_srow)
        ],
        out_specs=pl.BlockSpec((C, 128), lambda i: (i, 0)),
        out_shape=jax.ShapeDtypeStruct((s_rows, 128), jnp.float32),
    )


def kernel(first, second, emb1, emb2):
    b, f = first.shape
    emb_dim = emb1.shape[1]
    n_tab = emb1.shape[0]
    n_rows = b * f
    C = TR_C

    def remap(i):
        i = i.reshape(n_rows).astype(jnp.int32)
        return ((i & ~(4 * C - 1)) + 4 * (i & (C - 1))
                + ((i >> C.bit_length() - 1) & 3))

    idx1 = remap(first)
    idx2 = remap(second)
    tr_call = _build_transpose_call(n_tab, emb_dim)
    e1t, e2t = emb1.T, emb2.T
    s1 = tr_call(e1t, e1t, e1t, e1t)
    s2 = tr_call(e2t, e2t, e2t, e2t)
    e1 = s1.reshape(-1, emb_dim)
    e2 = s2.reshape(-1, emb_dim)
    sc_call = _build_sc_call(n_rows, emb_dim)
    out = sc_call(idx1, idx2, e1, e2)
    return out.reshape(b, f, emb_dim)


# speedup vs baseline: 2.1478x; 1.0162x over previous
import functools

import jax
import jax.numpy as jnp
from jax import lax
from jax.experimental import pallas as pl
from jax.experimental.pallas import tpu as pltpu
from jax.experimental.pallas import tpu_sc as plsc

LANES = 16


@functools.lru_cache(maxsize=None)
def _build_sc_call(n_rows: int, emb_dim: int):
    NW = 32
    per_w = n_rows // NW
    chunk = 128
    blk = 512
    gpb = blk // chunk
    n_blk = per_w // blk
    assert per_w * NW == n_rows and n_blk * blk == per_w and n_blk % 2 == 0

    mesh = plsc.VectorSubcoreMesh(core_axis_name="c", subcore_axis_name="s")

    @functools.partial(
        pl.kernel,
        out_type=jax.ShapeDtypeStruct((n_rows, emb_dim), jnp.float32),
        mesh=mesh,
        compiler_params=pltpu.CompilerParams(use_tc_tiling_on_sc=False),
        scratch_types=[
            pltpu.VMEM((per_w,), jnp.int32),
            pltpu.VMEM((per_w,), jnp.int32),
            pltpu.VMEM((blk, emb_dim), jnp.float32),
            pltpu.VMEM((blk, emb_dim), jnp.float32),
            pltpu.VMEM((blk, emb_dim), jnp.float32),
            pltpu.VMEM((blk, emb_dim), jnp.float32),
            pltpu.VMEM((blk, emb_dim), jnp.float32),
            pltpu.VMEM((blk, emb_dim), jnp.float32),
            pltpu.SemaphoreType.DMA,
            pltpu.SemaphoreType.DMA,
            pltpu.SemaphoreType.DMA,
            pltpu.SemaphoreType.DMA,
        ],
    )
    def sc_call(idx1_hbm, idx2_hbm, emb1_hbm, emb2_hbm, out_hbm,
                idx1_v, idx2_v, r1a, r1b, r2a, r2b, pa, pb,
                sg_a, sg_b, so_a, so_b):
        r1 = (r1a, r1b)
        r2 = (r2a, r2b)
        prod = (pa, pb)
        sg = (sg_a, sg_b)
        so = (so_a, so_b)

        wid = lax.axis_index("s") * 2 + lax.axis_index("c")
        base = wid * per_w
        pltpu.sync_copy(idx1_hbm.at[pl.ds(base, per_w)], idx1_v)
        pltpu.sync_copy(idx2_hbm.at[pl.ds(base, per_w)], idx2_v)

        def fire_gathers(j, slot):
            off = j * blk
            for g in range(gpb):
                o = off + g * chunk
                dst = pl.ds(g * chunk, chunk)
                pltpu.async_copy(
                    emb1_hbm.at[idx1_v.at[pl.ds(o, chunk)]],
                    r1[slot].at[dst], sg[slot])
                pltpu.async_copy(
                    emb2_hbm.at[idx2_v.at[pl.ds(o, chunk)]],
                    r2[slot].at[dst], sg[slot])

        def drain_gathers(slot):
            pltpu.make_async_copy(
                emb1_hbm.at[pl.ds(0, blk)], r1[slot], sg[slot]).wait()
            pltpu.make_async_copy(
                emb1_hbm.at[pl.ds(0, blk)], r2[slot], sg[slot]).wait()

        def drain_out(slot):
            pltpu.make_async_copy(
                prod[slot], out_hbm.at[pl.ds(0, blk)], so[slot]).wait()

        def multiply(slot):
            a, b, p = r1[slot], r2[slot], prod[slot]

            def mul_body(r, c):
                for h in range(emb_dim // LANES):
                    sl = pl.ds(h * LANES, LANES)
                    p[r, sl] = a[r, sl] * b[r, sl]
                return c

            lax.fori_loop(0, blk, mul_body, 0, unroll=8)

        fire_gathers(0, 0)
        fire_gathers(1, 1)

        def step(i, carry):
            for slot in range(2):
                j = 2 * i + slot
                drain_gathers(slot)

                @pl.when(i > 0)
                def _():
                    drain_out(slot)

                multiply(slot)

                @pl.when(j + 2 < n_blk)
                def _():
                    fire_gathers(j + 2, slot)

                pltpu.async_copy(
                    prod[slot], out_hbm.at[pl.ds(base + j * blk, blk)],
                    so[slot])
            return carry

        lax.fori_loop(0, n_blk // 2, step, 0)
        drain_out(0)
        drain_out(1)

    return sc_call


TR_C = 4096


@functools.lru_cache(maxsize=None)
def _build_transpose_call(n_tab: int, emb_dim: int):
    rows_per_srow = 128 // emb_dim
    C = TR_C
    grid = (n_tab + 4 * C - 1) // (4 * C)
    s_rows = grid * C
    max_blk = (n_tab + C - 1) // C - 1

    def body(*refs):
        in_refs, out_ref = refs[:rows_per_srow], refs[rows_per_srow]
        for j in range(rows_per_srow):
            out_ref[:, j * emb_dim:(j + 1) * emb_dim] = jnp.transpose(
                in_refs[j][...])

    return pl.pallas_call(
        body,
        grid=(grid,),
        in_specs=[
            pl.BlockSpec((emb_dim, C),
                         lambda i, j=j: (0, jnp.minimum(4 * i + j, max_blk)))
            for j in range(rows_per_srow)
        ],
        out_specs=pl.BlockSpec((C, 128), lambda i: (i, 0)),
        out_shape=jax.ShapeDtypeStruct((s_rows, 128), jnp.float32),
    )


def kernel(first, second, emb1, emb2):
    b, f = first.shape
    emb_dim = emb1.shape[1]
    n_tab = emb1.shape[0]
    n_rows = b * f
    C = TR_C

    def remap(i):
        i = i.reshape(n_rows).astype(jnp.int32)
        return ((i & ~(4 * C - 1)) + 4 * (i & (C - 1))
                + ((i >> C.bit_length() - 1) & 3))

    idx1 = remap(first)
    idx2 = remap(second)
    tr_call = _build_transpose_call(n_tab, emb_dim)
    e1t, e2t = emb1.T, emb2.T
    s1 = tr_call(e1t, e1t, e1t, e1t)
    s2 = tr_call(e2t, e2t, e2t, e2t)
    e1 = s1.reshape(-1, emb_dim)
    e2 = s2.reshape(-1, emb_dim)
    sc_call = _build_sc_call(n_rows, emb_dim)
    out = sc_call(idx1, idx2, e1, e2)
    return out.reshape(b, f, emb_dim)
